# baseline (speedup 1.0000x reference)
import jax
import jax.numpy as jnp
from jax import lax
from jax.experimental import pallas as pl
from jax.experimental.pallas import tpu as pltpu
from jax.experimental.pallas import tpu_sc as plsc

_DIM = 32
_BLK = 128
_NC = 2
_NS = 16
_NW = _NC * _NS


def _make_lookup(n_s):
  mesh = plsc.VectorSubcoreMesh(
      core_axis_name="c", subcore_axis_name="s",
      num_cores=_NC, num_subcores=_NS)
  n_tok = n_s * _BLK

  def body(table_hbm, idx_hbm, out_hbm, idx_v, cidx, rows, otp, gsem, ssem):
    jb = lax.axis_index("s") * _NC + lax.axis_index("c")
    pltpu.sync_copy(idx_hbm.at[pl.ds(jb * n_tok, n_tok)], idx_v)

    iota = lax.iota(jnp.int32, 16)
    iota_s = iota * n_s

    def extract(s, buf):
      for g in range(_BLK // 16):
        pv = iota_s + (g * 16 * n_s + s)
        cidx[buf, pl.ds(g * 16, 16)] = plsc.load_gather(idx_v, [pv])

    def fire_gather(buf):
      pltpu.async_copy(table_hbm.at[cidx.at[buf]], rows.at[buf],
                       gsem.at[buf])

    def wait_gather(buf):
      pltpu.make_async_copy(table_hbm.at[pl.ds(0, _BLK), :], rows.at[buf],
                            gsem.at[buf]).wait()

    def transpose(buf):
      r = rows.at[buf]
      for g in range(_BLK * _DIM // 16):
        k = g * 16
        tl_vec = iota + (k & 127)
        d_vec = jnp.full((16,), k >> 7, jnp.int32)
        v = plsc.load_gather(r, [tl_vec, d_vec])
        otp[buf, k >> 10, (k >> 7) & 7, pl.ds(k & 127, 16)] = v

    def fire_store(s, buf):
      pltpu.async_copy(otp.at[buf], out_hbm.at[s, :, jb], ssem.at[buf])

    def wait_store(s, buf):
      pltpu.make_async_copy(otp.at[buf], out_hbm.at[s, :, jb],
                            ssem.at[buf]).wait()

    extract(0, 0)
    fire_gather(0)

    def step(i, carry):
      for p in range(2):
        s = 2 * i + p

        @pl.when(s + 1 < n_s)
        def _():
          extract(s + 1, 1 - p)
          fire_gather(1 - p)

        wait_gather(p)

        @pl.when(s >= 2)
        def _():
          wait_store(s - 2, p)

        transpose(p)
        fire_store(s, p)
      return carry

    lax.fori_loop(0, n_s // 2, step, 0)
    wait_store(n_s - 2, 0)
    wait_store(n_s - 1, 1)

  return pl.kernel(
      body,
      out_type=jax.ShapeDtypeStruct((n_s, _DIM // 8, _NW, 8, _BLK),
                                    jnp.float32),
      mesh=mesh,
      scratch_types=[
          pltpu.VMEM((n_tok,), jnp.int32),
          pltpu.VMEM((2, _BLK), jnp.int32),
          pltpu.VMEM((2, _BLK, _DIM), jnp.float32),
          pltpu.VMEM((2, _DIM // 8, 8, _BLK), jnp.float32),
          pltpu.SemaphoreType.DMA((2,)),
          pltpu.SemaphoreType.DMA((2,)),
      ],
      compiler_params=pltpu.CompilerParams(use_tc_tiling_on_sc=False,
                                           needs_layout_passes=False),
  )


def kernel(x, table):
  r, s = x.shape
  idx = x.reshape(-1).astype(jnp.int32)
  out5 = _make_lookup(s)(table, idx)
  return out5.transpose(2, 4, 0, 1, 3).reshape(r, s, _DIM)


# speedup vs baseline: 1.1887x; 1.1887x over previous
import jax
import jax.numpy as jnp
from jax import lax
from jax.experimental import pallas as pl
from jax.experimental.pallas import tpu as pltpu
from jax.experimental.pallas import tpu_sc as plsc

_DIM = 32
_BLK = 128
_NC = 2
_NS = 16
_NW = _NC * _NS


def _make_lookup(n_s):
  mesh = plsc.VectorSubcoreMesh(
      core_axis_name="c", subcore_axis_name="s",
      num_cores=_NC, num_subcores=_NS)
  n_tok = n_s * _BLK

  def body(table_hbm, idx_hbm, out_hbm, idx_v, cidx, rows, otp, gsem, ssem):
    jb = lax.axis_index("s") * _NC + lax.axis_index("c")
    pltpu.sync_copy(idx_hbm.at[pl.ds(jb * n_tok, n_tok)], idx_v)

    iota = lax.iota(jnp.int32, 16)
    iota_s = iota * n_s

    def extract(s, buf):
      for g in range(_BLK // 16):
        pv = iota_s + (g * 16 * n_s + s)
        cidx[buf, pl.ds(g * 16, 16)] = plsc.load_gather(idx_v, [pv])

    def fire_gather(buf):
      pltpu.async_copy(table_hbm.at[cidx.at[buf]], rows.at[buf],
                       gsem.at[buf])

    def wait_gather(buf):
      pltpu.make_async_copy(table_hbm.at[pl.ds(0, _BLK), :], rows.at[buf],
                            gsem.at[buf]).wait()

    def transpose(buf):
      r = rows.at[buf]

      @plsc.parallel_loop(0, _BLK * _DIM // 16, unroll=8)
      def _(g):
        k = g * 16
        tl_vec = iota + (k & 127)
        d_vec = jnp.broadcast_to(k >> 7, (16,)).astype(jnp.int32)
        v = plsc.load_gather(r, [tl_vec, d_vec])
        otp[buf, k >> 10, (k >> 7) & 7, pl.ds(k & 127, 16)] = v

    def fire_store(s, buf):
      pltpu.async_copy(otp.at[buf], out_hbm.at[s, :, jb], ssem.at[buf])

    def wait_store(s, buf):
      pltpu.make_async_copy(otp.at[buf], out_hbm.at[s, :, jb],
                            ssem.at[buf]).wait()

    extract(0, 0)
    fire_gather(0)

    def step(i, carry):
      for p in range(2):
        s = 2 * i + p

        @pl.when(s + 1 < n_s)
        def _():
          extract(s + 1, 1 - p)
          fire_gather(1 - p)

        wait_gather(p)

        @pl.when(s >= 2)
        def _():
          wait_store(s - 2, p)

        transpose(p)
        fire_store(s, p)
      return carry

    lax.fori_loop(0, n_s // 2, step, 0)
    wait_store(n_s - 2, 0)
    wait_store(n_s - 1, 1)

  return pl.kernel(
      body,
      out_type=jax.ShapeDtypeStruct((n_s, _DIM // 8, _NW, 8, _BLK),
                                    jnp.float32),
      mesh=mesh,
      scratch_types=[
          pltpu.VMEM((n_tok,), jnp.int32),
          pltpu.VMEM((2, _BLK), jnp.int32),
          pltpu.VMEM((2, _BLK, _DIM), jnp.float32),
          pltpu.VMEM((2, _DIM // 8, 8, _BLK), jnp.float32),
          pltpu.SemaphoreType.DMA((2,)),
          pltpu.SemaphoreType.DMA((2,)),
      ],
      compiler_params=pltpu.CompilerParams(use_tc_tiling_on_sc=False,
                                           needs_layout_passes=False),
  )


def kernel(x, table):
  r, s = x.shape
  idx = x.reshape(-1).astype(jnp.int32)
  out5 = _make_lookup(s)(table, idx)
  return out5.transpose(2, 4, 0, 1, 3).reshape(r, s, _DIM)


# speedup vs baseline: 1.1979x; 1.0078x over previous
import jax
import jax.numpy as jnp
from jax import lax
from jax.experimental import pallas as pl
from jax.experimental.pallas import tpu as pltpu
from jax.experimental.pallas import tpu_sc as plsc

_DIM = 32
_BLK = 128
_NC = 2
_NS = 16
_NW = _NC * _NS


def _make_lookup(n_s):
  mesh = plsc.VectorSubcoreMesh(
      core_axis_name="c", subcore_axis_name="s",
      num_cores=_NC, num_subcores=_NS)
  n_tok = n_s * _BLK

  def body(table_hbm, idx_hbm, out_hbm, idx_v, cidx, rows, otp, gsem, ssem):
    jb = lax.axis_index("s") * _NC + lax.axis_index("c")
    pltpu.sync_copy(idx_hbm.at[pl.ds(jb * n_tok, n_tok)], idx_v)

    iota = lax.iota(jnp.int32, 16)
    iota_s = iota * n_s

    def extract(s, buf):
      for g in range(_BLK // 16):
        pv = iota_s + (g * 16 * n_s + s)
        cidx[buf, pl.ds(g * 16, 16)] = plsc.load_gather(idx_v, [pv])

    def fire_gather(buf):
      pltpu.async_copy(table_hbm.at[cidx.at[buf]], rows.at[buf],
                       gsem.at[buf])

    def wait_gather(buf):
      pltpu.make_async_copy(table_hbm.at[pl.ds(0, _BLK), :], rows.at[buf],
                            gsem.at[buf]).wait()

    def transpose(buf):
      r = rows.at[buf]
      d_vecs = [jnp.full((16,), d, jnp.int32) for d in range(_DIM)]

      @plsc.parallel_loop(0, _BLK, 16, unroll=2)
      def _(tl0):
        tl_vec = iota + tl0
        for d in range(_DIM):
          v = plsc.load_gather(r, [tl_vec, d_vecs[d]])
          otp[buf, d >> 3, d & 7, pl.ds(tl0, 16)] = v

    def fire_store(s, buf):
      pltpu.async_copy(otp.at[buf], out_hbm.at[s, :, jb], ssem.at[buf])

    def wait_store(s, buf):
      pltpu.make_async_copy(otp.at[buf], out_hbm.at[s, :, jb],
                            ssem.at[buf]).wait()

    extract(0, 0)
    fire_gather(0)

    def step(i, carry):
      for p in range(2):
        s = 2 * i + p

        @pl.when(s + 1 < n_s)
        def _():
          extract(s + 1, 1 - p)
          fire_gather(1 - p)

        wait_gather(p)

        @pl.when(s >= 2)
        def _():
          wait_store(s - 2, p)

        transpose(p)
        fire_store(s, p)
      return carry

    lax.fori_loop(0, n_s // 2, step, 0)
    wait_store(n_s - 2, 0)
    wait_store(n_s - 1, 1)

  return pl.kernel(
      body,
      out_type=jax.ShapeDtypeStruct((n_s, _DIM // 8, _NW, 8, _BLK),
                                    jnp.float32),
      mesh=mesh,
      scratch_types=[
          pltpu.VMEM((n_tok,), jnp.int32),
          pltpu.VMEM((2, _BLK), jnp.int32),
          pltpu.VMEM((2, _BLK, _DIM), jnp.float32),
          pltpu.VMEM((2, _DIM // 8, 8, _BLK), jnp.float32),
          pltpu.SemaphoreType.DMA((2,)),
          pltpu.SemaphoreType.DMA((2,)),
      ],
      compiler_params=pltpu.CompilerParams(use_tc_tiling_on_sc=False,
                                           needs_layout_passes=False),
  )


def kernel(x, table):
  r, s = x.shape
  idx = x.reshape(-1).astype(jnp.int32)
  out5 = _make_lookup(s)(table, idx)
  return out5.transpose(2, 4, 0, 1, 3).reshape(r, s, _DIM)
